# baseline (speedup 1.0000x reference)
import functools

import jax
import jax.numpy as jnp
import numpy as np
from jax.experimental import pallas as pl
from jax.experimental.pallas import tpu as pltpu
from jax.sharding import PartitionSpec as P

EPS = 1e-7

_N, _M, _D, _K, _H, _W = 16, 1, 8, 1024, 32, 32
_P = _N * _H * _W
_TP = 256
_GRID = _P // _TP

_R1 = (13, 15, 26, 6)
_R2 = (17, 29, 16, 24)


def _tf_round(x0, x1, r):
    x0 = x0 + x1
    x1 = (x1 << np.uint32(r)) | (x1 >> np.uint32(32 - r))
    x1 = x0 ^ x1
    return x0, x1


def _threefry_bits(k1, k2, counts):
    ks0, ks1 = k1, k2
    ks2 = ks0 ^ ks1 ^ np.uint32(0x1BD11BDA)
    x0 = ks0
    x1 = counts + ks1
    for r in _R1:
        x0, x1 = _tf_round(x0, x1, r)
    x0 = x0 + ks1
    x1 = x1 + (ks2 + np.uint32(1))
    for r in _R2:
        x0, x1 = _tf_round(x0, x1, r)
    x0 = x0 + ks2
    x1 = x1 + (ks0 + np.uint32(2))
    for r in _R1:
        x0, x1 = _tf_round(x0, x1, r)
    x0 = x0 + ks0
    x1 = x1 + (ks1 + np.uint32(3))
    for r in _R2:
        x0, x1 = _tf_round(x0, x1, r)
    x0 = x0 + ks1
    x1 = x1 + (ks2 + np.uint32(4))
    for r in _R1:
        x0, x1 = _tf_round(x0, x1, r)
    x0 = x0 + ks2
    x1 = x1 + (ks0 + np.uint32(5))
    return x0 ^ x1


def _bits_to_unit_float(bits):
    fb = (bits >> np.uint32(9)) | np.uint32(0x3F800000)
    return jax.lax.bitcast_convert_type(fb, jnp.float32) - jnp.float32(1.0)


def _quant_kernel(key_ref, t_ref, pb_ref, x_ref, cbt_ref, cb_ref, freq_ref,
                  out_ref, code_ref):
    i = pl.program_id(0)
    bits_log2 = jnp.float32(np.log2(_K))
    scale = jnp.float32(np.sqrt(_K))

    xt = x_ref[...]
    cbt = cbt_ref[...]
    x2 = jnp.sum(xt * xt, axis=1, keepdims=True)
    c2 = jnp.sum(cbt * cbt, axis=0, keepdims=True)
    inter = jnp.dot(xt, cbt, preferred_element_type=jnp.float32)
    dist = (x2 + c2) - jnp.float32(2.0) * inter
    t = jnp.maximum(t_ref[0, 0], jnp.float32(EPS))
    sneg = t * jnp.float32(-1.0 / 32.0)
    logit = dist * sneg

    freq = freq_ref[...]
    code_usage = jnp.clip(jnp.mean((freq > jnp.float32(EPS)).astype(jnp.float32)),
                          jnp.float32(0.0), jnp.float32(1.0))
    expo = -(bits_log2 - jnp.float32(1.0)) * code_usage * code_usage + bits_log2

    base = ((pb_ref[0, 0] + i * np.int32(_TP)) * np.int32(_K)).astype(jnp.int32)
    row = jax.lax.broadcasted_iota(jnp.int32, (_TP, _K), 0)
    col = jax.lax.broadcasted_iota(jnp.int32, (_TP, _K), 1)
    counts = (base + row * np.int32(_K) + col).astype(jnp.uint32)

    rbits = _threefry_bits(key_ref[0, 0], key_ref[0, 1], counts)
    r = _bits_to_unit_float(rbits)
    random_mask = (r ** expo) < freq
    logit = jnp.where(random_mask, logit - jnp.float32(1e9), logit)

    lmax = jnp.max(logit, axis=1, keepdims=True)
    code = jnp.min(jnp.where(logit == lmax, col, np.int32(_K)), axis=1)
    code_ref[...] = code.reshape(1, 1, _TP)

    ubits = _threefry_bits(key_ref[1, 0], key_ref[1, 1], counts)
    uflt = _bits_to_unit_float(ubits)
    u = jnp.maximum(jnp.float32(1e-20), uflt)
    gumbels = -jnp.log(-jnp.log(u))
    z = logit + gumbels
    zmax = jnp.max(z, axis=1, keepdims=True)
    idx = jnp.min(jnp.where(z == zmax, col, np.int32(_K)),
                  axis=1, keepdims=True)
    sample = (col == idx).astype(jnp.float32)

    out_ref[...] = jnp.dot(sample, cb_ref[...],
                           preferred_element_type=jnp.float32)


def _run_shard(keys, t2d, x2d_l, cbt, cb2d, freq2d, ndev):
    p_l, d = x2d_l.shape
    k = cb2d.shape[0]
    grid = p_l // _TP
    sid = jax.lax.axis_index("dp").astype(jnp.int32)
    pbase = (sid * np.int32(p_l)).reshape(1, 1)

    out2d, code3d = pl.pallas_call(
        _quant_kernel,
        grid=(grid,),
        in_specs=[
            pl.BlockSpec(memory_space=pltpu.MemorySpace.SMEM),
            pl.BlockSpec(memory_space=pltpu.MemorySpace.SMEM),
            pl.BlockSpec(memory_space=pltpu.MemorySpace.SMEM),
            pl.BlockSpec((_TP, d), lambda i: (i, 0)),
            pl.BlockSpec((d, k), lambda i: (0, 0)),
            pl.BlockSpec((k, d), lambda i: (0, 0)),
            pl.BlockSpec((1, k), lambda i: (0, 0)),
        ],
        out_specs=[
            pl.BlockSpec((_TP, d), lambda i: (i, 0)),
            pl.BlockSpec((1, 1, _TP), lambda i: (i, 0, 0)),
        ],
        out_shape=[
            jax.ShapeDtypeStruct((p_l, d), jnp.float32),
            jax.ShapeDtypeStruct((grid, 1, _TP), jnp.int32),
        ],
        compiler_params=pltpu.CompilerParams(
            dimension_semantics=("arbitrary",)),
    )(keys, t2d, pbase, x2d_l, cbt, cb2d, freq2d)
    return out2d, code3d


@jax.jit
def kernel(x, codebook, temperature, freqEMA):
    n, c, h, w = x.shape
    m, k, d = codebook.shape
    p = n * h * w

    x2d = x.reshape(n, m * d, h * w).transpose(0, 2, 1).reshape(p, m * d)
    cb2d = codebook.reshape(k, d)
    cbt = cb2d.T
    freq2d = freqEMA.reshape(1, k)
    t2d = temperature.reshape(1, 1)

    ku, kg = jax.random.split(jax.random.key(42))
    keys = jnp.stack([jax.random.key_data(ku),
                      jax.random.key_data(kg)]).astype(jnp.uint32)

    devs = jax.devices()
    ndev = 2 if len(devs) >= 2 else 1
    mesh = jax.sharding.Mesh(np.array(devs[:ndev]), ("dp",))
    rep = P()
    out2d, code3d = jax.shard_map(
        functools.partial(_run_shard, ndev=ndev),
        mesh=mesh,
        in_specs=(rep, rep, P("dp", None), rep, rep, rep),
        out_specs=(P("dp", None), P("dp", None, None)),
        check_vma=False,
    )(keys, t2d, x2d, cbt, cb2d, freq2d)

    out = out2d.reshape(n, h, w, m * d).transpose(0, 3, 1, 2)
    code = code3d.reshape(n, m, h, w)
    return out, code


# speedup vs baseline: 1.0392x; 1.0392x over previous
import functools

import jax
import jax.numpy as jnp
import numpy as np
from jax.experimental import pallas as pl
from jax.experimental.pallas import tpu as pltpu
from jax.sharding import PartitionSpec as P

EPS = 1e-7

_N, _M, _D, _K, _H, _W = 16, 1, 8, 1024, 32, 32
_P = _N * _H * _W
_TP = 256
_GRID = _P // _TP

_R1 = (13, 15, 26, 6)
_R2 = (17, 29, 16, 24)


def _tf_round(x0, x1, r):
    x0 = x0 + x1
    x1 = (x1 << np.uint32(r)) | (x1 >> np.uint32(32 - r))
    x1 = x0 ^ x1
    return x0, x1


def _threefry_bits(k1, k2, counts):
    ks0, ks1 = k1, k2
    ks2 = ks0 ^ ks1 ^ np.uint32(0x1BD11BDA)
    x0 = ks0
    x1 = counts + ks1
    for r in _R1:
        x0, x1 = _tf_round(x0, x1, r)
    x0 = x0 + ks1
    x1 = x1 + (ks2 + np.uint32(1))
    for r in _R2:
        x0, x1 = _tf_round(x0, x1, r)
    x0 = x0 + ks2
    x1 = x1 + (ks0 + np.uint32(2))
    for r in _R1:
        x0, x1 = _tf_round(x0, x1, r)
    x0 = x0 + ks0
    x1 = x1 + (ks1 + np.uint32(3))
    for r in _R2:
        x0, x1 = _tf_round(x0, x1, r)
    x0 = x0 + ks1
    x1 = x1 + (ks2 + np.uint32(4))
    for r in _R1:
        x0, x1 = _tf_round(x0, x1, r)
    x0 = x0 + ks2
    x1 = x1 + (ks0 + np.uint32(5))
    return x0 ^ x1


def _bits_to_unit_float(bits):
    fb = (bits >> np.uint32(9)) | np.uint32(0x3F800000)
    return jax.lax.bitcast_convert_type(fb, jnp.float32) - jnp.float32(1.0)


def _quant_kernel(key_ref, t_ref, pb_ref, x_ref, cbt_ref, cb_ref, freq_ref,
                  out_ref, code_ref):
    i = pl.program_id(0)
    bits_log2 = jnp.float32(np.log2(_K))
    scale = jnp.float32(np.sqrt(_K))

    xt = x_ref[...]
    cbt = cbt_ref[...]
    x2 = jnp.sum(xt * xt, axis=1, keepdims=True)
    c2 = jnp.sum(cbt * cbt, axis=0, keepdims=True)
    inter = jnp.dot(xt, cbt, preferred_element_type=jnp.float32)
    dist = (x2 + c2) - jnp.float32(2.0) * inter
    t = jnp.maximum(t_ref[0, 0], jnp.float32(EPS))
    sneg = t * jnp.float32(-1.0 / 32.0)
    logit = dist * sneg

    freq = freq_ref[...]
    code_usage = jnp.clip(jnp.mean((freq > jnp.float32(EPS)).astype(jnp.float32)),
                          jnp.float32(0.0), jnp.float32(1.0))
    expo = -(bits_log2 - jnp.float32(1.0)) * code_usage * code_usage + bits_log2

    base = ((pb_ref[0, 0] + i * np.int32(_TP)) * np.int32(_K)).astype(jnp.int32)
    row = jax.lax.broadcasted_iota(jnp.int32, (_TP, _K), 0)
    col = jax.lax.broadcasted_iota(jnp.int32, (_TP, _K), 1)
    counts = (base + row * np.int32(_K) + col).astype(jnp.uint32)

    rbits = _threefry_bits(key_ref[0, 0], key_ref[0, 1], counts)
    r = _bits_to_unit_float(rbits)
    random_mask = (r ** expo) < freq
    logit = jnp.where(random_mask, logit - jnp.float32(1e9), logit)

    lmax = jnp.max(logit, axis=1, keepdims=True)
    code = jnp.min(jnp.where(logit == lmax, col, np.int32(_K)), axis=1)
    code_ref[...] = code.reshape(1, 1, _TP)

    ubits = _threefry_bits(key_ref[1, 0], key_ref[1, 1], counts)
    uflt = _bits_to_unit_float(ubits)
    u = jnp.maximum(jnp.float32(1e-20), uflt)
    gumbels = -jnp.log(-jnp.log(u))
    z = logit + gumbels
    zmax = jnp.max(z, axis=1, keepdims=True)
    idx = jnp.min(jnp.where(z == zmax, col, np.int32(_K)),
                  axis=1, keepdims=True)
    sample = (col == idx).astype(jnp.float32)

    out_ref[...] = jnp.dot(sample, cb_ref[...],
                           preferred_element_type=jnp.float32)


def _run_shard(keys, t2d, x_l, cbt, cb2d, freq2d):
    n_l, c, h, w = x_l.shape
    k, d = cb2d.shape
    p_l = n_l * h * w
    x2d_l = x_l.reshape(n_l, c, h * w).transpose(0, 2, 1).reshape(p_l, c)
    grid = p_l // _TP
    sid = jax.lax.axis_index("dp").astype(jnp.int32)
    pbase = (sid * np.int32(p_l)).reshape(1, 1)

    out2d, code3d = pl.pallas_call(
        _quant_kernel,
        grid=(grid,),
        in_specs=[
            pl.BlockSpec(memory_space=pltpu.MemorySpace.SMEM),
            pl.BlockSpec(memory_space=pltpu.MemorySpace.SMEM),
            pl.BlockSpec(memory_space=pltpu.MemorySpace.SMEM),
            pl.BlockSpec((_TP, d), lambda i: (i, 0)),
            pl.BlockSpec((d, k), lambda i: (0, 0)),
            pl.BlockSpec((k, d), lambda i: (0, 0)),
            pl.BlockSpec((1, k), lambda i: (0, 0)),
        ],
        out_specs=[
            pl.BlockSpec((_TP, d), lambda i: (i, 0)),
            pl.BlockSpec((1, 1, _TP), lambda i: (i, 0, 0)),
        ],
        out_shape=[
            jax.ShapeDtypeStruct((p_l, d), jnp.float32),
            jax.ShapeDtypeStruct((grid, 1, _TP), jnp.int32),
        ],
        compiler_params=pltpu.CompilerParams(
            dimension_semantics=("arbitrary",)),
    )(keys, t2d, pbase, x2d_l, cbt, cb2d, freq2d)

    out_l = out2d.reshape(n_l, h, w, c).transpose(0, 3, 1, 2)
    code_l = code3d.reshape(n_l, 1, h, w)
    return out_l, code_l


@jax.jit
def kernel(x, codebook, temperature, freqEMA):
    n, c, h, w = x.shape
    m, k, d = codebook.shape

    cb2d = codebook.reshape(k, d)
    cbt = cb2d.T
    freq2d = freqEMA.reshape(1, k)
    t2d = temperature.reshape(1, 1)

    ku, kg = jax.random.split(jax.random.key(42))
    keys = jnp.stack([jax.random.key_data(ku),
                      jax.random.key_data(kg)]).astype(jnp.uint32)

    devs = jax.devices()
    ndev = 2 if len(devs) >= 2 else 1
    mesh = jax.sharding.Mesh(np.array(devs[:ndev]), ("dp",))
    rep = P()
    out, code = jax.shard_map(
        _run_shard,
        mesh=mesh,
        in_specs=(rep, rep, P("dp", None, None, None), rep, rep, rep),
        out_specs=(P("dp", None, None, None), P("dp", None, None, None)),
        check_vma=False,
    )(keys, t2d, x, cbt, cb2d, freq2d)
    return out, code


# speedup vs baseline: 1.1564x; 1.1128x over previous
import jax
import jax.numpy as jnp
import numpy as np
from jax.experimental import pallas as pl
from jax.experimental.pallas import tpu as pltpu

EPS = 1e-7

_N, _M, _D, _K, _H, _W = 16, 1, 8, 1024, 32, 32
_P = _N * _H * _W
_TP = 256
_GRID = _P // _TP

_R1 = (13, 15, 26, 6)
_R2 = (17, 29, 16, 24)


def _tf_round(x0, x1, r):
    x0 = x0 + x1
    x1 = (x1 << np.uint32(r)) | (x1 >> np.uint32(32 - r))
    x1 = x0 ^ x1
    return x0, x1


def _threefry_bits(k1, k2, counts_plus_ks1):
    ks0, ks1 = k1, k2
    ks2 = ks0 ^ ks1 ^ np.uint32(0x1BD11BDA)
    x0 = ks0
    x1 = counts_plus_ks1
    for r in _R1:
        x0, x1 = _tf_round(x0, x1, r)
    x0 = x0 + ks1
    x1 = x1 + (ks2 + np.uint32(1))
    for r in _R2:
        x0, x1 = _tf_round(x0, x1, r)
    x0 = x0 + ks2
    x1 = x1 + (ks0 + np.uint32(2))
    for r in _R1:
        x0, x1 = _tf_round(x0, x1, r)
    x0 = x0 + ks0
    x1 = x1 + (ks1 + np.uint32(3))
    for r in _R2:
        x0, x1 = _tf_round(x0, x1, r)
    x0 = x0 + ks1
    x1 = x1 + (ks2 + np.uint32(4))
    for r in _R1:
        x0, x1 = _tf_round(x0, x1, r)
    x0 = x0 + ks2
    x1 = x1 + (ks0 + np.uint32(5))
    return x0 ^ x1


def _bits_to_unit_float(bits):
    fb = (bits >> np.uint32(9)) | np.uint32(0x3F800000)
    return jax.lax.bitcast_convert_type(fb, jnp.float32) - jnp.float32(1.0)


def _quant_kernel(key_ref, t_ref, pb_ref, x_ref, cbt_ref, cb_ref, freq_ref,
                  out_ref, code_ref):
    i = pl.program_id(0)
    bits_log2 = jnp.float32(np.log2(_K))

    xt = x_ref[...]
    cbt = cbt_ref[...]
    x2 = jnp.sum(xt * xt, axis=1, keepdims=True)
    c2 = jnp.sum(cbt * cbt, axis=0, keepdims=True)
    inter = jnp.dot(xt, cbt, preferred_element_type=jnp.float32)
    dist = (x2 + c2) - jnp.float32(2.0) * inter
    t = jnp.maximum(t_ref[0, 0], jnp.float32(EPS))
    sneg = t * jnp.float32(-1.0 / 32.0)
    logit = dist * sneg

    freq = freq_ref[...]
    code_usage = jnp.clip(jnp.mean((freq > jnp.float32(EPS)).astype(jnp.float32)),
                          jnp.float32(0.0), jnp.float32(1.0))
    expo = -(bits_log2 - jnp.float32(1.0)) * code_usage * code_usage + bits_log2

    base = ((pb_ref[0, 0] + i * np.int32(_TP)) * np.int32(_K)).astype(jnp.uint32)
    row = jax.lax.broadcasted_iota(jnp.int32, (_TP, _K), 0)
    col = jax.lax.broadcasted_iota(jnp.int32, (_TP, _K), 1)
    counts0 = (row * np.int32(_K) + col).astype(jnp.uint32)

    rbits = _threefry_bits(key_ref[0, 0], key_ref[0, 1],
                           counts0 + (base + key_ref[0, 1]))
    r = _bits_to_unit_float(rbits)
    random_mask = (r ** expo) < freq
    logit = jnp.where(random_mask, logit - jnp.float32(1e9), logit)

    lmax = jnp.max(logit, axis=1, keepdims=True)
    code = jnp.min(jnp.where(logit == lmax, col, np.int32(_K)), axis=1)
    code_ref[...] = code.reshape(1, 1, _TP)

    ubits = _threefry_bits(key_ref[1, 0], key_ref[1, 1],
                           counts0 + (base + key_ref[1, 1]))
    uflt = _bits_to_unit_float(ubits)
    u = jnp.maximum(jnp.float32(1e-20), uflt)
    gumbels = -jnp.log(-jnp.log(u))
    z = logit + gumbels
    zmax = jnp.max(z, axis=1, keepdims=True)
    idx = jnp.min(jnp.where(z == zmax, col, np.int32(_K)),
                  axis=1, keepdims=True)
    sample = (col == idx).astype(jnp.float32)

    out_ref[...] = jnp.dot(sample, cb_ref[...],
                           preferred_element_type=jnp.float32)


def _run(keys, t2d, x_l, cbt, cb2d, freq2d):
    n_l, c, h, w = x_l.shape
    k, d = cb2d.shape
    p_l = n_l * h * w
    x2d_l = x_l.reshape(n_l, c, h * w).transpose(0, 2, 1).reshape(p_l, c)
    grid = p_l // _TP
    pbase = jnp.zeros((1, 1), jnp.int32)

    out2d, code3d = pl.pallas_call(
        _quant_kernel,
        grid=(grid,),
        in_specs=[
            pl.BlockSpec(memory_space=pltpu.MemorySpace.SMEM),
            pl.BlockSpec(memory_space=pltpu.MemorySpace.SMEM),
            pl.BlockSpec(memory_space=pltpu.MemorySpace.SMEM),
            pl.BlockSpec((_TP, d), lambda i: (i, 0)),
            pl.BlockSpec((d, k), lambda i: (0, 0)),
            pl.BlockSpec((k, d), lambda i: (0, 0)),
            pl.BlockSpec((1, k), lambda i: (0, 0)),
        ],
        out_specs=[
            pl.BlockSpec((_TP, d), lambda i: (i, 0)),
            pl.BlockSpec((1, 1, _TP), lambda i: (i, 0, 0)),
        ],
        out_shape=[
            jax.ShapeDtypeStruct((p_l, d), jnp.float32),
            jax.ShapeDtypeStruct((grid, 1, _TP), jnp.int32),
        ],
        compiler_params=pltpu.CompilerParams(
            dimension_semantics=("arbitrary",)),
    )(keys, t2d, pbase, x2d_l, cbt, cb2d, freq2d)

    out_l = out2d.reshape(n_l, h, w, c).transpose(0, 3, 1, 2)
    code_l = code3d.reshape(n_l, 1, h, w)
    return out_l, code_l


@jax.jit
def kernel(x, codebook, temperature, freqEMA):
    n, c, h, w = x.shape
    m, k, d = codebook.shape

    cb2d = codebook.reshape(k, d)
    cbt = cb2d.T
    freq2d = freqEMA.reshape(1, k)
    t2d = temperature.reshape(1, 1)

    ku, kg = jax.random.split(jax.random.key(42))
    keys = jnp.stack([jax.random.key_data(ku),
                      jax.random.key_data(kg)]).astype(jnp.uint32)

    out, code = _run(keys, t2d, x, cbt, cb2d, freq2d)
    return out, code


# speedup vs baseline: 1.1876x; 1.0269x over previous
import jax
import jax.numpy as jnp
import numpy as np
from jax.experimental import pallas as pl
from jax.experimental.pallas import tpu as pltpu

EPS = 1e-7

_N, _M, _D, _K, _H, _W = 16, 1, 8, 1024, 32, 32
_P = _N * _H * _W
_TP = 512
_GRID = _P // _TP

_R1 = (13, 15, 26, 6)
_R2 = (17, 29, 16, 24)


def _tf_round(x0, x1, r):
    x0 = x0 + x1
    x1 = (x1 << np.uint32(r)) | (x1 >> np.uint32(32 - r))
    x1 = x0 ^ x1
    return x0, x1


def _threefry_bits(k1, k2, counts_plus_ks1):
    ks0, ks1 = k1, k2
    ks2 = ks0 ^ ks1 ^ np.uint32(0x1BD11BDA)
    x0 = ks0
    x1 = counts_plus_ks1
    for r in _R1:
        x0, x1 = _tf_round(x0, x1, r)
    x0 = x0 + ks1
    x1 = x1 + (ks2 + np.uint32(1))
    for r in _R2:
        x0, x1 = _tf_round(x0, x1, r)
    x0 = x0 + ks2
    x1 = x1 + (ks0 + np.uint32(2))
    for r in _R1:
        x0, x1 = _tf_round(x0, x1, r)
    x0 = x0 + ks0
    x1 = x1 + (ks1 + np.uint32(3))
    for r in _R2:
        x0, x1 = _tf_round(x0, x1, r)
    x0 = x0 + ks1
    x1 = x1 + (ks2 + np.uint32(4))
    for r in _R1:
        x0, x1 = _tf_round(x0, x1, r)
    x0 = x0 + ks2
    x1 = x1 + (ks0 + np.uint32(5))
    return x0 ^ x1


def _bits_to_unit_float(bits):
    fb = (bits >> np.uint32(9)) | np.uint32(0x3F800000)
    return jax.lax.bitcast_convert_type(fb, jnp.float32) - jnp.float32(1.0)


def _quant_kernel(key_ref, t_ref, pb_ref, x_ref, cbt_ref, cb_ref, freq_ref,
                  out_ref, code_ref):
    i = pl.program_id(0)
    bits_log2 = jnp.float32(np.log2(_K))

    xt = x_ref[...]
    cbt = cbt_ref[...]
    x2 = jnp.sum(xt * xt, axis=1, keepdims=True)
    c2 = jnp.sum(cbt * cbt, axis=0, keepdims=True)
    inter = jnp.dot(xt, cbt, preferred_element_type=jnp.float32)
    dist = (x2 + c2) - jnp.float32(2.0) * inter
    t = jnp.maximum(t_ref[0, 0], jnp.float32(EPS))
    sneg = t * jnp.float32(-1.0 / 32.0)
    logit = dist * sneg

    freq = freq_ref[...]
    code_usage = jnp.clip(jnp.mean((freq > jnp.float32(EPS)).astype(jnp.float32)),
                          jnp.float32(0.0), jnp.float32(1.0))
    expo = -(bits_log2 - jnp.float32(1.0)) * code_usage * code_usage + bits_log2

    base = ((pb_ref[0, 0] + i * np.int32(_TP)) * np.int32(_K)).astype(jnp.uint32)
    row = jax.lax.broadcasted_iota(jnp.int32, (_TP, _K), 0)
    col = jax.lax.broadcasted_iota(jnp.int32, (_TP, _K), 1)
    counts0 = (row * np.int32(_K) + col).astype(jnp.uint32)

    rbits = _threefry_bits(key_ref[0, 0], key_ref[0, 1],
                           counts0 + (base + key_ref[0, 1]))
    r = _bits_to_unit_float(rbits)
    random_mask = (r ** expo) < freq
    logit = jnp.where(random_mask, logit - jnp.float32(1e9), logit)

    lmax = jnp.max(logit, axis=1, keepdims=True)
    code = jnp.min(jnp.where(logit == lmax, col, np.int32(_K)), axis=1)
    code_ref[...] = code.reshape(1, 1, _TP)

    ubits = _threefry_bits(key_ref[1, 0], key_ref[1, 1],
                           counts0 + (base + key_ref[1, 1]))
    uflt = _bits_to_unit_float(ubits)
    u = jnp.maximum(jnp.float32(1e-20), uflt)
    gumbels = -jnp.log(-jnp.log(u))
    z = logit + gumbels
    zmax = jnp.max(z, axis=1, keepdims=True)
    idx = jnp.min(jnp.where(z == zmax, col, np.int32(_K)),
                  axis=1, keepdims=True)
    sample = (col == idx).astype(jnp.float32)

    out_ref[...] = jnp.dot(sample, cb_ref[...],
                           preferred_element_type=jnp.float32)


def _run(keys, t2d, x_l, cbt, cb2d, freq2d):
    n_l, c, h, w = x_l.shape
    k, d = cb2d.shape
    p_l = n_l * h * w
    x2d_l = x_l.reshape(n_l, c, h * w).transpose(0, 2, 1).reshape(p_l, c)
    grid = p_l // _TP
    pbase = jnp.zeros((1, 1), jnp.int32)

    out2d, code3d = pl.pallas_call(
        _quant_kernel,
        grid=(grid,),
        in_specs=[
            pl.BlockSpec(memory_space=pltpu.MemorySpace.SMEM),
            pl.BlockSpec(memory_space=pltpu.MemorySpace.SMEM),
            pl.BlockSpec(memory_space=pltpu.MemorySpace.SMEM),
            pl.BlockSpec((_TP, d), lambda i: (i, 0)),
            pl.BlockSpec((d, k), lambda i: (0, 0)),
            pl.BlockSpec((k, d), lambda i: (0, 0)),
            pl.BlockSpec((1, k), lambda i: (0, 0)),
        ],
        out_specs=[
            pl.BlockSpec((_TP, d), lambda i: (i, 0)),
            pl.BlockSpec((1, 1, _TP), lambda i: (i, 0, 0)),
        ],
        out_shape=[
            jax.ShapeDtypeStruct((p_l, d), jnp.float32),
            jax.ShapeDtypeStruct((grid, 1, _TP), jnp.int32),
        ],
        compiler_params=pltpu.CompilerParams(
            dimension_semantics=("arbitrary",)),
    )(keys, t2d, pbase, x2d_l, cbt, cb2d, freq2d)

    out_l = out2d.reshape(n_l, h, w, c).transpose(0, 3, 1, 2)
    code_l = code3d.reshape(n_l, 1, h, w)
    return out_l, code_l


@jax.jit
def kernel(x, codebook, temperature, freqEMA):
    n, c, h, w = x.shape
    m, k, d = codebook.shape

    cb2d = codebook.reshape(k, d)
    cbt = cb2d.T
    freq2d = freqEMA.reshape(1, k)
    t2d = temperature.reshape(1, 1)

    ku, kg = jax.random.split(jax.random.key(42))
    keys = jnp.stack([jax.random.key_data(ku),
                      jax.random.key_data(kg)]).astype(jnp.uint32)

    out, code = _run(keys, t2d, x, cbt, cb2d, freq2d)
    return out, code
